# baseline (speedup 1.0000x reference)
import functools
import jax
import jax.numpy as jnp
from jax import lax
from jax.experimental import pallas as pl
from jax.experimental.pallas import tpu as pltpu
from jax.experimental.pallas import tpu_sc as plsc

N_CLASS = 1000000
EMBED_DIM = 32
BATCH = 16384
FIELDS = 26

B = BATCH * FIELDS
NC = 2
NS = 16
NW = NC * NS
L_PER_W = B // NW
CHUNK = 208
N_CHUNK = L_PER_W // CHUNK
STREAM = CHUNK // 2
N_GRP = CHUNK // 16


def _fire(xi_hbm, t128_hbm, cbase, xv, gv, rows_v, sem):
    pltpu.sync_copy(xi_hbm.at[pl.ds(cbase, CHUNK)], xv)
    for g in range(N_GRP):
        gv[pl.ds(g * 16, 16)] = xv[pl.ds(g * 16, 16)] >> 2
    copies = []
    for j in range(2):
        copies.append(
            pltpu.async_copy(
                t128_hbm.at[gv.at[pl.ds(j * STREAM, STREAM)]],
                rows_v.at[pl.ds(j * STREAM, STREAM)],
                sem,
            )
        )
    return copies


def _drain_extract(out_hbm, cbase, copies, xv, rows_v, out_f):
    for c in copies:
        c.wait()
    iota = lax.iota(jnp.int32, 16)
    for g in range(N_GRP):
        x_vec = xv[pl.ds(g * 16, 16)]
        o_vec = (x_vec & 3) << 5
        j_vec = iota + (g * 16)
        d_vec = j_vec * EMBED_DIM
        for c in range(EMBED_DIM):
            vals = plsc.load_gather(rows_v, [j_vec, o_vec + c])
            plsc.store_scatter(out_f, [d_vec + c], vals)
    pltpu.sync_copy(out_f, out_hbm.at[pl.ds(cbase * EMBED_DIM, CHUNK * EMBED_DIM)])


def _emb_body(
    xi_hbm, t128_hbm, out_hbm,
    xv0, gv0, rows0, outf0, xv1, gv1, rows1, outf1, sem0, sem1,
):
    wid = lax.axis_index("s") * NC + lax.axis_index("c")
    lbase = wid * L_PER_W

    def pair_body(ci, carry):
        c0 = lbase + (2 * ci) * CHUNK
        c1 = lbase + (2 * ci + 1) * CHUNK
        cp0 = _fire(xi_hbm, t128_hbm, c0, xv0, gv0, rows0, sem0)
        cp1 = _fire(xi_hbm, t128_hbm, c1, xv1, gv1, rows1, sem1)
        _drain_extract(out_hbm, c0, cp0, xv0, rows0, outf0)
        _drain_extract(out_hbm, c1, cp1, xv1, rows1, outf1)
        return carry

    lax.fori_loop(0, N_CHUNK // 2, pair_body, 0, unroll=False)


@jax.jit
def kernel(X, table):
    xi = X.astype(jnp.int32).reshape(B)
    t128 = table.reshape(N_CLASS // 4, 128)
    mesh = plsc.VectorSubcoreMesh(core_axis_name="c", subcore_axis_name="s")
    f = functools.partial(
        pl.kernel,
        mesh=mesh,
        out_type=jax.ShapeDtypeStruct((B * EMBED_DIM,), jnp.float32),
        compiler_params=pltpu.CompilerParams(needs_layout_passes=False),
        scratch_types=[
            pltpu.VMEM((CHUNK,), jnp.int32),
            pltpu.VMEM((CHUNK,), jnp.int32),
            pltpu.VMEM((CHUNK, 128), jnp.float32),
            pltpu.VMEM((CHUNK * EMBED_DIM,), jnp.float32),
            pltpu.VMEM((CHUNK,), jnp.int32),
            pltpu.VMEM((CHUNK,), jnp.int32),
            pltpu.VMEM((CHUNK, 128), jnp.float32),
            pltpu.VMEM((CHUNK * EMBED_DIM,), jnp.float32),
            pltpu.SemaphoreType.DMA,
            pltpu.SemaphoreType.DMA,
        ],
    )(_emb_body)
    out_flat = f(xi, t128)
    return out_flat.reshape(BATCH, FIELDS, EMBED_DIM)


# speedup vs baseline: 1.3356x; 1.3356x over previous
import functools
import jax
import jax.numpy as jnp
from jax import lax
from jax.experimental import pallas as pl
from jax.experimental.pallas import tpu as pltpu
from jax.experimental.pallas import tpu_sc as plsc

N_CLASS = 1000000
EMBED_DIM = 32
BATCH = 16384
FIELDS = 26

B = BATCH * FIELDS
NC = 2
NS = 16
NW = NC * NS
S_PER_W = BATCH // NW
S_PER_CHUNK = 8
CHUNK = S_PER_CHUNK * FIELDS
N_CHUNK = S_PER_W // S_PER_CHUNK
STREAM = CHUNK // 2


def _fire(x_hbm, t128_hbm, s0, xv, gv, rows_v, sem):
    iota = lax.iota(jnp.int32, 16)
    pltpu.sync_copy(x_hbm.at[pl.ds(s0, S_PER_CHUNK)], xv)
    for si in range(S_PER_CHUNK):
        v1 = xv[si, pl.ds(0, 16)] >> 2
        v2 = xv[si, pl.ds(10, 16)] >> 2
        plsc.store_scatter(gv, [iota + (FIELDS * si)], v1)
        plsc.store_scatter(gv, [iota + (FIELDS * si + 10)], v2)
    copies = []
    for j in range(2):
        copies.append(
            pltpu.async_copy(
                t128_hbm.at[gv.at[pl.ds(j * STREAM, STREAM)]],
                rows_v.at[pl.ds(j * STREAM, STREAM)],
                sem,
            )
        )
    return copies


def _drain_extract(out_hbm, s0, copies, xv, rows_v, out_v, osem):
    for c in copies:
        c.wait()
    iota = lax.iota(jnp.int32, 16)
    for si in range(S_PER_CHUNK):
        x1 = xv[si, pl.ds(0, 16)]
        o1 = (x1 & 3) << 5
        x2 = xv[si, pl.ds(10, 16)]
        o2 = (x2 & 3) << 5
        for f in range(FIELDS):
            if f < 16:
                osp = jnp.take(o1, jnp.full((16,), f, jnp.int32))
            else:
                osp = jnp.take(o2, jnp.full((16,), f - 10, jnp.int32))
            j = FIELDS * si + f
            jv = jnp.full((16,), j, jnp.int32)
            out_v[j, pl.ds(0, 16)] = plsc.load_gather(
                rows_v, [jv, osp + iota])
            out_v[j, pl.ds(16, 16)] = plsc.load_gather(
                rows_v, [jv, osp + iota + 16])
    ocopies = []
    for si in range(S_PER_CHUNK):
        ocopies.append(
            pltpu.async_copy(
                out_v.at[pl.ds(FIELDS * si, FIELDS)],
                out_hbm.at[s0 + si],
                osem,
            )
        )
    for c in ocopies:
        c.wait()


def _emb_body(
    x_hbm, t128_hbm, out_hbm,
    xv0, gv0, rows0, outv0, xv1, gv1, rows1, outv1, sem0, sem1, osem,
):
    wid = lax.axis_index("s") * NC + lax.axis_index("c")
    sbase = wid * S_PER_W

    def pair_body(ci, carry):
        s0 = sbase + (2 * ci) * S_PER_CHUNK
        s1 = sbase + (2 * ci + 1) * S_PER_CHUNK
        cp0 = _fire(x_hbm, t128_hbm, s0, xv0, gv0, rows0, sem0)
        cp1 = _fire(x_hbm, t128_hbm, s1, xv1, gv1, rows1, sem1)
        _drain_extract(out_hbm, s0, cp0, xv0, rows0, outv0, osem)
        _drain_extract(out_hbm, s1, cp1, xv1, rows1, outv1, osem)
        return carry

    lax.fori_loop(0, N_CHUNK // 2, pair_body, 0, unroll=False)


@jax.jit
def kernel(X, table):
    xi = X.astype(jnp.int32)
    t128 = table.reshape(N_CLASS // 4, 128)
    mesh = plsc.VectorSubcoreMesh(core_axis_name="c", subcore_axis_name="s")
    f = functools.partial(
        pl.kernel,
        mesh=mesh,
        out_type=jax.ShapeDtypeStruct((BATCH, FIELDS, EMBED_DIM), jnp.float32),
        compiler_params=pltpu.CompilerParams(needs_layout_passes=False),
        scratch_types=[
            pltpu.VMEM((S_PER_CHUNK, FIELDS), jnp.int32),
            pltpu.VMEM((CHUNK,), jnp.int32),
            pltpu.VMEM((CHUNK, 128), jnp.float32),
            pltpu.VMEM((CHUNK, EMBED_DIM), jnp.float32),
            pltpu.VMEM((S_PER_CHUNK, FIELDS), jnp.int32),
            pltpu.VMEM((CHUNK,), jnp.int32),
            pltpu.VMEM((CHUNK, 128), jnp.float32),
            pltpu.VMEM((CHUNK, EMBED_DIM), jnp.float32),
            pltpu.SemaphoreType.DMA,
            pltpu.SemaphoreType.DMA,
            pltpu.SemaphoreType.DMA,
        ],
    )(_emb_body)
    return f(xi, t128)


# speedup vs baseline: 1.6854x; 1.2619x over previous
import functools
import jax
import jax.numpy as jnp
from jax import lax
from jax.experimental import pallas as pl
from jax.experimental.pallas import tpu as pltpu
from jax.experimental.pallas import tpu_sc as plsc

N_CLASS = 1000000
EMBED_DIM = 32
BATCH = 16384
FIELDS = 26

NC = 2
NS = 16
NW = NC * NS
S_PER_W = BATCH // NW
S_PER_CHUNK = 16
CHUNK = S_PER_CHUNK * FIELDS
N_CHUNK = S_PER_W // S_PER_CHUNK
STREAM = 104
N_STREAM = CHUNK // STREAM


def _fire(x_hbm, table_hbm, s0, xv, gv, rows_v, sem):
    iota = lax.iota(jnp.int32, 16)
    pltpu.sync_copy(x_hbm.at[pl.ds(s0, S_PER_CHUNK)], xv)
    for si in range(S_PER_CHUNK):
        v1 = xv[si, pl.ds(0, 16)]
        v2 = xv[si, pl.ds(10, 16)]
        plsc.store_scatter(gv, [iota + (FIELDS * si)], v1)
        plsc.store_scatter(gv, [iota + (FIELDS * si + 10)], v2)
    copies = []
    for j in range(N_STREAM):
        copies.append(
            pltpu.async_copy(
                table_hbm.at[gv.at[pl.ds(j * STREAM, STREAM)]],
                rows_v.at[pl.ds(j * STREAM, STREAM)],
                sem,
            )
        )
    return copies


def _drain_store(out_hbm, s0, copies, rows_v, osem):
    for c in copies:
        c.wait()
    ocopies = []
    for si in range(S_PER_CHUNK):
        ocopies.append(
            pltpu.async_copy(
                rows_v.at[pl.ds(FIELDS * si, FIELDS)],
                out_hbm.at[s0 + si],
                osem,
            )
        )
    return ocopies


def _emb_body(
    x_hbm, table_hbm, out_hbm,
    xv0, gv0, rows0, xv1, gv1, rows1, sem0, sem1, osem,
):
    wid = lax.axis_index("s") * NC + lax.axis_index("c")
    sbase = wid * S_PER_W

    def pair_body(ci, carry):
        s0 = sbase + (2 * ci) * S_PER_CHUNK
        s1 = sbase + (2 * ci + 1) * S_PER_CHUNK
        cp0 = _fire(x_hbm, table_hbm, s0, xv0, gv0, rows0, sem0)
        cp1 = _fire(x_hbm, table_hbm, s1, xv1, gv1, rows1, sem1)
        ocp0 = _drain_store(out_hbm, s0, cp0, rows0, osem)
        ocp1 = _drain_store(out_hbm, s1, cp1, rows1, osem)
        for c in ocp0 + ocp1:
            c.wait()
        return carry

    lax.fori_loop(0, N_CHUNK // 2, pair_body, 0, unroll=False)


@jax.jit
def kernel(X, table):
    xi = X.astype(jnp.int32)
    mesh = plsc.VectorSubcoreMesh(core_axis_name="c", subcore_axis_name="s")
    f = functools.partial(
        pl.kernel,
        mesh=mesh,
        out_type=jax.ShapeDtypeStruct((BATCH, FIELDS, EMBED_DIM), jnp.float32),
        compiler_params=pltpu.CompilerParams(
            needs_layout_passes=False, use_tc_tiling_on_sc=False
        ),
        scratch_types=[
            pltpu.VMEM((S_PER_CHUNK, FIELDS), jnp.int32),
            pltpu.VMEM((CHUNK,), jnp.int32),
            pltpu.VMEM((CHUNK, EMBED_DIM), jnp.float32),
            pltpu.VMEM((S_PER_CHUNK, FIELDS), jnp.int32),
            pltpu.VMEM((CHUNK,), jnp.int32),
            pltpu.VMEM((CHUNK, EMBED_DIM), jnp.float32),
            pltpu.SemaphoreType.DMA,
            pltpu.SemaphoreType.DMA,
            pltpu.SemaphoreType.DMA,
        ],
    )(_emb_body)
    return f(xi, table)
